# baseline (speedup 1.0000x reference)
import jax
import jax.numpy as jnp
from jax import lax
from jax.experimental import pallas as pl
from jax.experimental.pallas import tpu as pltpu
from jax.experimental.pallas import tpu_sc as plsc

FACTOR_REG = 0.0005
BIAS_REG = 0.01

B = 16384
K = 64
NC = 2
NS = 16
NW = NC * NS
BPW = B // NW
CHUNK = 128
NCHUNK = BPW // CHUNK


def _sc_body(u_r, i_r, j_r, ue_hbm, ie_hbm, ib_hbm,
             xuij_out, l2_out,
             idx_u, idx_i, idx_j, rows_u, rows_i, rows_j,
             ibv_buf, jbv_buf, xuij_v, l2_v, sem):
    wid = lax.axis_index("c") * NS + lax.axis_index("s")

    pltpu.sync_copy(u_r.at[pl.ds(wid * NCHUNK, NCHUNK)], idx_u)
    pltpu.sync_copy(i_r.at[pl.ds(wid * NCHUNK, NCHUNK)], idx_i)
    pltpu.sync_copy(j_r.at[pl.ds(wid * NCHUNK, NCHUNK)], idx_j)

    copies = []
    for c in range(NCHUNK):
        sl = pl.ds(c * CHUNK, CHUNK)
        copies.append(pltpu.async_copy(ue_hbm.at[idx_u.at[c]], rows_u.at[sl], sem))
        copies.append(pltpu.async_copy(ie_hbm.at[idx_i.at[c]], rows_i.at[sl], sem))
        copies.append(pltpu.async_copy(ie_hbm.at[idx_j.at[c]], rows_j.at[sl], sem))
        copies.append(pltpu.async_copy(ib_hbm.at[idx_i.at[c]], ibv_buf.at[sl], sem))
        copies.append(pltpu.async_copy(ib_hbm.at[idx_j.at[c]], jbv_buf.at[sl], sem))
    for cp in copies:
        cp.wait()

    lane = lax.iota(jnp.int32, 16)
    zf = jnp.zeros((16,), jnp.float32)

    def group(gg, carry):
        l2f, l2ib, l2jb = carry
        rb = gg * 16
        xvec = zf
        for r in range(16):
            row = rb + r
            pu = [rows_u[row, pl.ds(q * 16, 16)] for q in range(K // 16)]
            pi = [rows_i[row, pl.ds(q * 16, 16)] for q in range(K // 16)]
            pj = [rows_j[row, pl.ds(q * 16, 16)] for q in range(K // 16)]
            di = zf
            dj = zf
            for q in range(K // 16):
                di = di + pu[q] * pi[q]
                dj = dj + pu[q] * pj[q]
                l2f = l2f + pu[q] * pu[q]
                l2f = l2f + pi[q] * pi[q]
                l2f = l2f + pj[q] * pj[q]
            d = jnp.sum(di - dj)
            xvec = jnp.where(lane == r, d, xvec)
        ibv = ibv_buf[pl.ds(rb, 16)]
        jbv = jbv_buf[pl.ds(rb, 16)]
        x = xvec + (ibv - jbv)
        xuij_v[pl.ds(rb, 16)] = x
        l2ib = l2ib + ibv * ibv
        l2jb = l2jb + jbv * jbv
        return l2f, l2ib, l2jb

    l2f, l2ib, l2jb = lax.fori_loop(0, BPW // 16, group, (zf, zf, zf))
    l2_v[...] = (jnp.float32(FACTOR_REG) * l2f
                 + jnp.float32(BIAS_REG) * l2ib
                 + jnp.float32(BIAS_REG / 10.0) * l2jb)

    pltpu.sync_copy(xuij_v, xuij_out.at[pl.ds(wid * BPW, BPW)])
    pltpu.sync_copy(l2_v, l2_out.at[wid])


_sc_call = pl.kernel(
    _sc_body,
    out_type=(
        jax.ShapeDtypeStruct((B,), jnp.float32),
        jax.ShapeDtypeStruct((NW, 16), jnp.float32),
    ),
    mesh=plsc.VectorSubcoreMesh(core_axis_name="c", subcore_axis_name="s"),
    compiler_params=pltpu.CompilerParams(
        needs_layout_passes=False, use_tc_tiling_on_sc=False),
    scratch_types=[
        pltpu.VMEM((NCHUNK, CHUNK), jnp.int32),
        pltpu.VMEM((NCHUNK, CHUNK), jnp.int32),
        pltpu.VMEM((NCHUNK, CHUNK), jnp.int32),
        pltpu.VMEM((BPW, K), jnp.float32),
        pltpu.VMEM((BPW, K), jnp.float32),
        pltpu.VMEM((BPW, K), jnp.float32),
        pltpu.VMEM((BPW,), jnp.float32),
        pltpu.VMEM((BPW,), jnp.float32),
        pltpu.VMEM((BPW,), jnp.float32),
        pltpu.VMEM((16,), jnp.float32),
        pltpu.SemaphoreType.DMA,
    ],
)


def _tc_body(x_ref, l2_ref, loss_ref, auc_ref):
    x = x_ref[...]
    l2 = jnp.sum(l2_ref[...])
    logsig = jnp.sum(jnp.log(jax.nn.sigmoid(x)))
    auc = jnp.sum((x > 0).astype(jnp.float32))
    loss_ref[0, 0] = l2 - logsig / jnp.float32(B)
    auc_ref[0, 0] = auc / jnp.float32(B)


_tc_call = pl.pallas_call(
    _tc_body,
    out_shape=(
        jax.ShapeDtypeStruct((1, 1), jnp.float32),
        jax.ShapeDtypeStruct((1, 1), jnp.float32),
    ),
    out_specs=(
        pl.BlockSpec(memory_space=pltpu.SMEM),
        pl.BlockSpec(memory_space=pltpu.SMEM),
    ),
)


def kernel(u, i, j, user_emb_w, item_emb_w, item_b):
    u_r = u.astype(jnp.int32).reshape(NW * NCHUNK, CHUNK)
    i_r = i.astype(jnp.int32).reshape(NW * NCHUNK, CHUNK)
    j_r = j.astype(jnp.int32).reshape(NW * NCHUNK, CHUNK)
    ib_flat = item_b.reshape(-1)
    xuij, l2p = _sc_call(u_r, i_r, j_r, user_emb_w, item_emb_w, ib_flat)
    loss, auc = _tc_call(xuij.reshape(128, 128), l2p)
    return (loss[0, 0], auc[0, 0])
